# baseline (speedup 1.0000x reference)
import functools

import jax
import jax.numpy as jnp
from jax import lax
from jax.experimental import pallas as pl
from jax.experimental.pallas import tpu as pltpu, tpu_sc as plsc

VOCAB = 30522
HIDDEN = 768
MAX_POS = 512
BATCH = 1024
SEQ = 512
EPS = 1e-12

L = 16
NSLICE = HIDDEN // L
NW = 32
SEQ_PER_W = BATCH // NW
C = 16
NCHUNK = SEQ // C
NPAIR = SEQ_PER_W // 2


def _sc_body(ids_hbm, word_hbm, pos_hbm, type_hbm, gamma_hbm, beta_hbm,
             out_hbm, idx_all, rows0, rows1, ob0, ob1, pos_v, type_v,
             gamma_v, beta_v, acc_v, acc2_v, mean_v, rstd_v,
             sg0, sg1, ss0, ss1, si):
    wid = lax.axis_index("s") * 2 + lax.axis_index("c")
    row0 = wid * SEQ_PER_W
    rows = (rows0, rows1)
    obs = (ob0, ob1)
    sgs = (sg0, sg1)
    sss = (ss0, ss1)

    pltpu.sync_copy(type_hbm.at[pl.ds(0, HIDDEN)], type_v)
    pltpu.sync_copy(gamma_hbm, gamma_v)
    pltpu.sync_copy(beta_hbm, beta_v)

    def pass1(rbuf, obuf):
        @plsc.parallel_loop(0, C, unroll=2)
        def tok(t):
            a = [jnp.zeros((L,), jnp.float32) for _ in range(4)]
            q = [jnp.zeros((L,), jnp.float32) for _ in range(4)]
            for j in range(NSLICE):
                sl = pl.ds(j * L, L)
                v = rbuf[t, sl] + pos_v[t, sl]
                obuf[t, sl] = v
                a[j % 4] = a[j % 4] + v
                q[j % 4] = q[j % 4] + v * v
            acc_v[pl.ds(t * L, L)] = (a[0] + a[1]) + (a[2] + a[3])
            acc2_v[pl.ds(t * L, L)] = (q[0] + q[1]) + (q[2] + q[3])

    def stats():
        @plsc.parallel_loop(0, C // L)
        def grp(g):
            toks = g * L + lax.iota(jnp.int32, L)
            s1 = jnp.zeros((L,), jnp.float32)
            s2 = jnp.zeros((L,), jnp.float32)
            base16 = toks * L
            for l in range(L):
                s1 = s1 + plsc.load_gather(acc_v, [base16 + l])
                s2 = s2 + plsc.load_gather(acc2_v, [base16 + l])
            mean = s1 * (1.0 / HIDDEN)
            x = jnp.maximum(s2 * (1.0 / HIDDEN) - mean * mean, 0.0) + EPS
            bits = plsc.bitcast(x, jnp.int32)
            y = plsc.bitcast(jnp.int32(0x5F3759DF) - (bits >> 1), jnp.float32)
            xh = x * 0.5
            for _ in range(3):
                y = y * (1.5 - xh * y * y)
            mean_v[pl.ds(g * L, L)] = mean
            rstd_v[pl.ds(g * L, L)] = y

    def pass2(obuf):
        @plsc.parallel_loop(0, C, unroll=2)
        def tok(t):
            tt = jnp.full((L,), t, jnp.int32)
            mv = plsc.load_gather(mean_v, [tt])
            rs = plsc.load_gather(rstd_v, [tt])
            for j in range(NSLICE):
                sl = pl.ds(j * L, L)
                v = (obuf[t, sl] - mv) * rs
                obuf[t, sl] = v * gamma_v[sl] + beta_v[sl]

    def pair_body(p, carry):
        ci = p // NPAIR
        pi = p % NPAIR
        s0 = ci * C

        @pl.when(pi == 0)
        def _chunk_top():
            pltpu.sync_copy(pos_hbm.at[pl.ds(s0, C)], pos_v)

            @plsc.parallel_loop(0, C)
            def add_type(t):
                for j in range(NSLICE):
                    sl = pl.ds(j * L, L)
                    pos_v[t, sl] = pos_v[t, sl] + type_v[sl]
            for i in range(SEQ_PER_W):
                pltpu.async_copy(
                    ids_hbm.at[pl.ds((row0 + i) * SEQ + s0, C)],
                    idx_all.at[pl.ds(i * C, C)], si)
            for i in range(SEQ_PER_W):
                pltpu.make_async_copy(
                    ids_hbm.at[pl.ds((row0 + i) * SEQ + s0, C)],
                    idx_all.at[pl.ds(i * C, C)], si).wait()
            pltpu.async_copy(word_hbm.at[idx_all.at[pl.ds(0, C)]], rows0, sg0)

        for b in (0, 1):
            i = 2 * pi + b
            rbuf, obuf, sg, ss = rows[b], obs[b], sgs[b], sss[b]

            not_first = jnp.logical_or(ci > 0, i >= 2)

            @pl.when(not_first)
            def _wait_store(obuf=obuf, ss=ss):
                pltpu.make_async_copy(obuf, out_hbm.at[pl.ds(0, C)],
                                      ss).wait()

            @pl.when(i + 1 < SEQ_PER_W)
            def _issue_gather(b=b, i=i):
                pltpu.async_copy(word_hbm.at[idx_all.at[pl.ds((i + 1) * C, C)]],
                                 rows[1 - b], sgs[1 - b])

            pltpu.make_async_copy(word_hbm.at[idx_all.at[pl.ds(i * C, C)]], rbuf,
                                  sg).wait()
            pass1(rbuf, obuf)
            stats()
            pass2(obuf)
            base = (row0 + i) * SEQ + s0
            pltpu.async_copy(obuf, out_hbm.at[pl.ds(base, C)], ss)
        return carry

    lax.fori_loop(0, NCHUNK * NPAIR, pair_body, 0)
    pltpu.make_async_copy(ob0, out_hbm.at[pl.ds(0, C)], ss0).wait()
    pltpu.make_async_copy(ob1, out_hbm.at[pl.ds(0, C)], ss1).wait()


@jax.jit
def kernel(input_ids, word_emb, pos_emb, type_emb, gamma, beta):
    mesh = plsc.VectorSubcoreMesh(core_axis_name="c", subcore_axis_name="s")
    k = functools.partial(
        pl.kernel,
        out_type=jax.ShapeDtypeStruct((BATCH * SEQ, HIDDEN), jnp.float32),
        mesh=mesh,
        compiler_params=pltpu.CompilerParams(needs_layout_passes=False),
        scratch_types=[
            pltpu.VMEM((SEQ_PER_W * C,), jnp.int32),
            pltpu.VMEM((C, HIDDEN), jnp.float32),
            pltpu.VMEM((C, HIDDEN), jnp.float32),
            pltpu.VMEM((C, HIDDEN), jnp.float32),
            pltpu.VMEM((C, HIDDEN), jnp.float32),
            pltpu.VMEM((C, HIDDEN), jnp.float32),
            pltpu.VMEM((HIDDEN,), jnp.float32),
            pltpu.VMEM((HIDDEN,), jnp.float32),
            pltpu.VMEM((HIDDEN,), jnp.float32),
            pltpu.VMEM((C * L,), jnp.float32),
            pltpu.VMEM((C * L,), jnp.float32),
            pltpu.VMEM((C,), jnp.float32),
            pltpu.VMEM((C,), jnp.float32),
            pltpu.SemaphoreType.DMA,
            pltpu.SemaphoreType.DMA,
            pltpu.SemaphoreType.DMA,
            pltpu.SemaphoreType.DMA,
            pltpu.SemaphoreType.DMA,
        ],
    )(_sc_body)
    out = k(input_ids.reshape(-1).astype(jnp.int32), word_emb, pos_emb,
            type_emb.reshape(-1), gamma, beta)
    return out.reshape(BATCH, SEQ, HIDDEN)


# speedup vs baseline: 2.8241x; 2.8241x over previous
import functools

import jax
import jax.numpy as jnp
from jax import lax
from jax.experimental import pallas as pl
from jax.experimental.pallas import tpu as pltpu, tpu_sc as plsc

VOCAB = 30522
HIDDEN = 768
MAX_POS = 512
BATCH = 1024
SEQ = 512
EPS = 1e-12

NW = 32
TOK_PER_W = BATCH * SEQ // NW
CG = 64
NB = TOK_PER_W // CG


def _sc_gather_body(ids_hbm, word_hbm, out_hbm, idx_v, r0, r1,
                    sg0, sg1, ss0, ss1):
    wid = lax.axis_index("s") * 2 + lax.axis_index("c")
    base0 = wid * TOK_PER_W
    rows = (r0, r1)
    sgs = (sg0, sg1)
    sss = (ss0, ss1)

    pltpu.sync_copy(ids_hbm.at[pl.ds(base0, TOK_PER_W)], idx_v)
    pltpu.async_copy(word_hbm.at[idx_v.at[pl.ds(0, CG)]], r0, sg0)

    def pair(p, carry):
        for b in (0, 1):
            i = 2 * p + b
            pltpu.make_async_copy(
                word_hbm.at[idx_v.at[pl.ds(i * CG, CG)]], rows[b],
                sgs[b]).wait()

            pltpu.async_copy(rows[b], out_hbm.at[pl.ds(base0 + i * CG, CG)],
                             sss[b])

            @pl.when(jnp.logical_and(i >= 1, i + 1 < NB))
            def _issue_gather(b=b, i=i):
                pltpu.make_async_copy(rows[1 - b],
                                      out_hbm.at[pl.ds(0, CG)],
                                      sss[1 - b]).wait()
                pltpu.async_copy(
                    word_hbm.at[idx_v.at[pl.ds((i + 1) * CG, CG)]],
                    rows[1 - b], sgs[1 - b])

            @pl.when(i == 0)
            def _issue_gather0(b=b, i=i):
                pltpu.async_copy(
                    word_hbm.at[idx_v.at[pl.ds((i + 1) * CG, CG)]],
                    rows[1 - b], sgs[1 - b])
        return carry

    lax.fori_loop(0, NB // 2, pair, 0)
    pltpu.make_async_copy(r0, out_hbm.at[pl.ds(0, CG)], ss0).wait()
    pltpu.make_async_copy(r1, out_hbm.at[pl.ds(0, CG)], ss1).wait()


def _tc_ln_body(x_ref, pos_ref, type_ref, gamma_ref, beta_ref, o_ref):
    x = x_ref[...] + pos_ref[...] + type_ref[0:1, :]
    m = jnp.mean(x, axis=-1, keepdims=True)
    d = x - m
    var = jnp.mean(d * d, axis=-1, keepdims=True)
    o = d * jax.lax.rsqrt(var + EPS)
    o_ref[...] = o * gamma_ref[0:1, :] + beta_ref[0:1, :]


@jax.jit
def kernel(input_ids, word_emb, pos_emb, type_emb, gamma, beta):
    mesh = plsc.VectorSubcoreMesh(core_axis_name="c", subcore_axis_name="s")
    gathered = functools.partial(
        pl.kernel,
        out_type=jax.ShapeDtypeStruct((BATCH * SEQ, HIDDEN), jnp.float32),
        mesh=mesh,
        compiler_params=pltpu.CompilerParams(needs_layout_passes=False),
        scratch_types=[
            pltpu.VMEM((TOK_PER_W,), jnp.int32),
            pltpu.VMEM((CG, HIDDEN), jnp.float32),
            pltpu.VMEM((CG, HIDDEN), jnp.float32),
            pltpu.SemaphoreType.DMA,
            pltpu.SemaphoreType.DMA,
            pltpu.SemaphoreType.DMA,
            pltpu.SemaphoreType.DMA,
        ],
    )(_sc_gather_body)(input_ids.reshape(-1).astype(jnp.int32), word_emb)

    out = pl.pallas_call(
        _tc_ln_body,
        grid=(BATCH,),
        in_specs=[
            pl.BlockSpec((SEQ, HIDDEN), lambda i: (i, 0)),
            pl.BlockSpec((SEQ, HIDDEN), lambda i: (0, 0)),
            pl.BlockSpec((2, HIDDEN), lambda i: (0, 0)),
            pl.BlockSpec((1, HIDDEN), lambda i: (0, 0)),
            pl.BlockSpec((1, HIDDEN), lambda i: (0, 0)),
        ],
        out_specs=pl.BlockSpec((SEQ, HIDDEN), lambda i: (i, 0)),
        out_shape=jax.ShapeDtypeStruct((BATCH * SEQ, HIDDEN), jnp.float32),
    )(gathered, pos_emb, type_emb, gamma.reshape(1, HIDDEN),
      beta.reshape(1, HIDDEN))
    return out.reshape(BATCH, SEQ, HIDDEN)


# speedup vs baseline: 3.0589x; 1.0832x over previous
import functools

import jax
import jax.numpy as jnp
from jax import lax
from jax.experimental import pallas as pl
from jax.experimental.pallas import tpu as pltpu, tpu_sc as plsc

VOCAB = 30522
HIDDEN = 768
MAX_POS = 512
BATCH = 1024
SEQ = 512
EPS = 1e-12

NW = 32
NCH = 4
CHB = BATCH // NCH
CHTOK = CHB * SEQ
TOK_PER_W = CHTOK // NW
CG = 64
NB = TOK_PER_W // CG


def _sc_gather_body(ids_hbm, word_hbm, out_hbm, idx_v, r0, r1,
                    sg0, sg1, ss0, ss1):
    wid = lax.axis_index("s") * 2 + lax.axis_index("c")
    base0 = wid * TOK_PER_W
    rows = (r0, r1)
    sgs = (sg0, sg1)
    sss = (ss0, ss1)

    pltpu.sync_copy(ids_hbm.at[pl.ds(base0, TOK_PER_W)], idx_v)
    pltpu.async_copy(word_hbm.at[idx_v.at[pl.ds(0, CG)]], r0, sg0)

    def pair(p, carry):
        for b in (0, 1):
            i = 2 * p + b
            pltpu.make_async_copy(
                word_hbm.at[idx_v.at[pl.ds(i * CG, CG)]], rows[b],
                sgs[b]).wait()

            pltpu.async_copy(rows[b], out_hbm.at[pl.ds(base0 + i * CG, CG)],
                             sss[b])

            @pl.when(jnp.logical_and(i >= 1, i + 1 < NB))
            def _issue_gather(b=b, i=i):
                pltpu.make_async_copy(rows[1 - b],
                                      out_hbm.at[pl.ds(0, CG)],
                                      sss[1 - b]).wait()
                pltpu.async_copy(
                    word_hbm.at[idx_v.at[pl.ds((i + 1) * CG, CG)]],
                    rows[1 - b], sgs[1 - b])

            @pl.when(i == 0)
            def _issue_gather0(b=b, i=i):
                pltpu.async_copy(
                    word_hbm.at[idx_v.at[pl.ds((i + 1) * CG, CG)]],
                    rows[1 - b], sgs[1 - b])
        return carry

    lax.fori_loop(0, NB // 2, pair, 0)
    pltpu.make_async_copy(r0, out_hbm.at[pl.ds(0, CG)], ss0).wait()
    pltpu.make_async_copy(r1, out_hbm.at[pl.ds(0, CG)], ss1).wait()


def _sc_gather(ids_chunk, word_emb):
    mesh = plsc.VectorSubcoreMesh(core_axis_name="c", subcore_axis_name="s")
    return functools.partial(
        pl.kernel,
        out_type=jax.ShapeDtypeStruct((CHTOK, HIDDEN), jnp.float32),
        mesh=mesh,
        compiler_params=pltpu.CompilerParams(needs_layout_passes=False),
        scratch_types=[
            pltpu.VMEM((TOK_PER_W,), jnp.int32),
            pltpu.VMEM((CG, HIDDEN), jnp.float32),
            pltpu.VMEM((CG, HIDDEN), jnp.float32),
            pltpu.SemaphoreType.DMA,
            pltpu.SemaphoreType.DMA,
            pltpu.SemaphoreType.DMA,
            pltpu.SemaphoreType.DMA,
        ],
    )(_sc_gather_body)(ids_chunk, word_emb)


def _tc_ln_first(x_ref, pos_ref, type_ref, gamma_ref, beta_ref, o_ref):
    x = x_ref[...] + pos_ref[...] + type_ref[0:1, :]
    m = jnp.mean(x, axis=-1, keepdims=True)
    d = x - m
    var = jnp.mean(d * d, axis=-1, keepdims=True)
    o = d * jax.lax.rsqrt(var + EPS)
    o_ref[...] = o * gamma_ref[0:1, :] + beta_ref[0:1, :]


def _tc_ln_acc(prev_ref, x_ref, pos_ref, type_ref, gamma_ref, beta_ref,
               o_ref):
    del prev_ref
    _tc_ln_first(x_ref, pos_ref, type_ref, gamma_ref, beta_ref, o_ref)


@jax.jit
def kernel(input_ids, word_emb, pos_emb, type_emb, gamma, beta):
    ids_flat = input_ids.reshape(-1).astype(jnp.int32)
    gm = gamma.reshape(1, HIDDEN)
    bt = beta.reshape(1, HIDDEN)

    gathered = [
        _sc_gather(lax.slice(ids_flat, (c * CHTOK,), ((c + 1) * CHTOK,)),
                   word_emb)
        for c in range(NCH)
    ]

    common_in = [
        pl.BlockSpec((SEQ, HIDDEN), lambda i: (i, 0)),
        pl.BlockSpec((SEQ, HIDDEN), lambda i: (0, 0)),
        pl.BlockSpec((2, HIDDEN), lambda i: (0, 0)),
        pl.BlockSpec((1, HIDDEN), lambda i: (0, 0)),
        pl.BlockSpec((1, HIDDEN), lambda i: (0, 0)),
    ]
    out_shape = jax.ShapeDtypeStruct((BATCH * SEQ, HIDDEN), jnp.float32)

    out = pl.pallas_call(
        _tc_ln_first,
        grid=(CHB,),
        in_specs=common_in,
        out_specs=pl.BlockSpec((SEQ, HIDDEN), lambda i: (i, 0)),
        out_shape=out_shape,
    )(gathered[0], pos_emb, type_emb, gm, bt)

    for c in range(1, NCH):
        out = pl.pallas_call(
            _tc_ln_acc,
            grid=(CHB,),
            in_specs=[pl.BlockSpec(memory_space=pl.ANY)] + common_in,
            out_specs=pl.BlockSpec((SEQ, HIDDEN),
                                   lambda i, c=c: (c * CHB + i, 0)),
            out_shape=out_shape,
            input_output_aliases={0: 0},
        )(out, gathered[c], pos_emb, type_emb, gm, bt)

    return out.reshape(BATCH, SEQ, HIDDEN)


# speedup vs baseline: 3.8849x; 1.2700x over previous
import functools

import jax
import jax.numpy as jnp
from jax import lax
from jax.experimental import pallas as pl
from jax.experimental.pallas import tpu as pltpu, tpu_sc as plsc

VOCAB = 30522
HIDDEN = 768
MAX_POS = 512
BATCH = 1024
SEQ = 512
EPS = 1e-12

NW = 32
NCH = 4
CHB = BATCH // NCH
CHTOK = CHB * SEQ
TOK_PER_W = CHTOK // NW
HP = HIDDEN // 2
CG = 64
NB = TOK_PER_W // CG


def _sc_gather_body(ids_hbm, word_hbm, out_hbm, idx_v, r0, r1,
                    sg0, sg1, ss0, ss1):
    wid = lax.axis_index("s") * 2 + lax.axis_index("c")
    base0 = wid * TOK_PER_W
    rows = (r0, r1)
    sgs = (sg0, sg1)
    sss = (ss0, ss1)

    pltpu.sync_copy(ids_hbm.at[pl.ds(base0, TOK_PER_W)], idx_v)
    pltpu.async_copy(word_hbm.at[idx_v.at[pl.ds(0, CG)]], r0, sg0)

    def pair(p, carry):
        for b in (0, 1):
            i = 2 * p + b
            pltpu.make_async_copy(
                word_hbm.at[idx_v.at[pl.ds(i * CG, CG)]], rows[b],
                sgs[b]).wait()

            pltpu.async_copy(rows[b], out_hbm.at[pl.ds(base0 + i * CG, CG)],
                             sss[b])

            @pl.when(jnp.logical_and(i >= 1, i + 1 < NB))
            def _issue_gather(b=b, i=i):
                pltpu.make_async_copy(rows[1 - b],
                                      out_hbm.at[pl.ds(0, CG)],
                                      sss[1 - b]).wait()
                pltpu.async_copy(
                    word_hbm.at[idx_v.at[pl.ds((i + 1) * CG, CG)]],
                    rows[1 - b], sgs[1 - b])

            @pl.when(i == 0)
            def _issue_gather0(b=b, i=i):
                pltpu.async_copy(
                    word_hbm.at[idx_v.at[pl.ds((i + 1) * CG, CG)]],
                    rows[1 - b], sgs[1 - b])
        return carry

    lax.fori_loop(0, NB // 2, pair, 0)
    pltpu.make_async_copy(r0, out_hbm.at[pl.ds(0, CG)], ss0).wait()
    pltpu.make_async_copy(r1, out_hbm.at[pl.ds(0, CG)], ss1).wait()


def _sc_gather(ids_chunk, word_emb):
    mesh = plsc.VectorSubcoreMesh(core_axis_name="c", subcore_axis_name="s")
    return functools.partial(
        pl.kernel,
        out_type=jax.ShapeDtypeStruct((CHTOK, HP), jnp.int32),
        mesh=mesh,
        compiler_params=pltpu.CompilerParams(needs_layout_passes=False),
        scratch_types=[
            pltpu.VMEM((TOK_PER_W,), jnp.int32),
            pltpu.VMEM((CG, HP), jnp.int32),
            pltpu.VMEM((CG, HP), jnp.int32),
            pltpu.SemaphoreType.DMA,
            pltpu.SemaphoreType.DMA,
            pltpu.SemaphoreType.DMA,
            pltpu.SemaphoreType.DMA,
        ],
    )(_sc_gather_body)(ids_chunk, word_emb)


def _tc_ln_first(x_ref, pos_ref, type_ref, gamma_ref, beta_ref, o_ref):
    u = x_ref[...]
    fe = jax.lax.bitcast_convert_type(u << 16, jnp.float32)
    fo = jax.lax.bitcast_convert_type(u & jnp.int32(-65536), jnp.float32)
    x = jnp.concatenate([fe, fo], axis=-1) + pos_ref[...] + type_ref[0:1, :]
    m = jnp.mean(x, axis=-1, keepdims=True)
    d = x - m
    var = jnp.mean(d * d, axis=-1, keepdims=True)
    o = d * jax.lax.rsqrt(var + EPS)
    o_ref[...] = o * gamma_ref[0:1, :] + beta_ref[0:1, :]


def _tc_ln_acc(prev_ref, x_ref, pos_ref, type_ref, gamma_ref, beta_ref,
               o_ref):
    del prev_ref
    _tc_ln_first(x_ref, pos_ref, type_ref, gamma_ref, beta_ref, o_ref)


@jax.jit
def kernel(input_ids, word_emb, pos_emb, type_emb, gamma, beta):
    ids_flat = input_ids.reshape(-1).astype(jnp.int32)
    gm = gamma.reshape(1, HIDDEN)
    bt = beta.reshape(1, HIDDEN)

    w16 = word_emb.astype(jnp.bfloat16)
    word_packed = jax.lax.bitcast_convert_type(
        jnp.stack([w16[:, :HP], w16[:, HP:]], axis=-1), jnp.int32)
    gathered = [
        _sc_gather(lax.slice(ids_flat, (c * CHTOK,), ((c + 1) * CHTOK,)),
                   word_packed)
        for c in range(NCH)
    ]

    common_in = [
        pl.BlockSpec((SEQ, HP), lambda i: (i, 0)),
        pl.BlockSpec((SEQ, HIDDEN), lambda i: (0, 0)),
        pl.BlockSpec((2, HIDDEN), lambda i: (0, 0)),
        pl.BlockSpec((1, HIDDEN), lambda i: (0, 0)),
        pl.BlockSpec((1, HIDDEN), lambda i: (0, 0)),
    ]
    out_shape = jax.ShapeDtypeStruct((BATCH * SEQ, HIDDEN), jnp.float32)

    out = pl.pallas_call(
        _tc_ln_first,
        grid=(CHB,),
        in_specs=common_in,
        out_specs=pl.BlockSpec((SEQ, HIDDEN), lambda i: (i, 0)),
        out_shape=out_shape,
    )(gathered[0], pos_emb, type_emb, gm, bt)

    for c in range(1, NCH):
        out = pl.pallas_call(
            _tc_ln_acc,
            grid=(CHB,),
            in_specs=[pl.BlockSpec(memory_space=pl.ANY)] + common_in,
            out_specs=pl.BlockSpec((SEQ, HIDDEN),
                                   lambda i, c=c: (c * CHB + i, 0)),
            out_shape=out_shape,
            input_output_aliases={0: 0},
        )(out, gathered[c], pos_emb, type_emb, gm, bt)

    return out.reshape(BATCH, SEQ, HIDDEN)


# speedup vs baseline: 4.3698x; 1.1248x over previous
import functools

import jax
import jax.numpy as jnp
from jax import lax
from jax.experimental import pallas as pl
from jax.experimental.pallas import tpu as pltpu, tpu_sc as plsc

VOCAB = 30522
HIDDEN = 768
MAX_POS = 512
BATCH = 1024
SEQ = 512
EPS = 1e-12

NW = 32
NCH = 4
CHB = BATCH // NCH
CHTOK = CHB * SEQ
TOK_PER_W = CHTOK // NW
HP = HIDDEN // 2
CG = 64
NB = TOK_PER_W // CG


def _sc_gather_body(ids_hbm, word_hbm, out_hbm, idx_v, r0, r1,
                    sg0, sg1, ss0, ss1):
    wid = lax.axis_index("s") * 2 + lax.axis_index("c")
    base0 = wid * TOK_PER_W
    rows = (r0, r1)
    sgs = (sg0, sg1)
    sss = (ss0, ss1)

    pltpu.sync_copy(ids_hbm.at[pl.ds(base0, TOK_PER_W)], idx_v)
    pltpu.async_copy(word_hbm.at[idx_v.at[pl.ds(0, CG)]], r0, sg0)

    def pair(p, carry):
        for b in (0, 1):
            i = 2 * p + b
            pltpu.make_async_copy(
                word_hbm.at[idx_v.at[pl.ds(i * CG, CG)]], rows[b],
                sgs[b]).wait()

            pltpu.async_copy(rows[b], out_hbm.at[pl.ds(base0 + i * CG, CG)],
                             sss[b])

            @pl.when(jnp.logical_and(i >= 1, i + 1 < NB))
            def _issue_gather(b=b, i=i):
                pltpu.make_async_copy(rows[1 - b],
                                      out_hbm.at[pl.ds(0, CG)],
                                      sss[1 - b]).wait()
                pltpu.async_copy(
                    word_hbm.at[idx_v.at[pl.ds((i + 1) * CG, CG)]],
                    rows[1 - b], sgs[1 - b])

            @pl.when(i == 0)
            def _issue_gather0(b=b, i=i):
                pltpu.async_copy(
                    word_hbm.at[idx_v.at[pl.ds((i + 1) * CG, CG)]],
                    rows[1 - b], sgs[1 - b])
        return carry

    lax.fori_loop(0, NB // 2, pair, 0)
    pltpu.make_async_copy(r0, out_hbm.at[pl.ds(0, CG)], ss0).wait()
    pltpu.make_async_copy(r1, out_hbm.at[pl.ds(0, CG)], ss1).wait()


def _sc_gather(ids_chunk, word_emb):
    mesh = plsc.VectorSubcoreMesh(core_axis_name="c", subcore_axis_name="s")
    return functools.partial(
        pl.kernel,
        out_type=jax.ShapeDtypeStruct((CHTOK, HP), jnp.int32),
        mesh=mesh,
        compiler_params=pltpu.CompilerParams(needs_layout_passes=False),
        scratch_types=[
            pltpu.VMEM((TOK_PER_W,), jnp.int32),
            pltpu.VMEM((CG, HP), jnp.int32),
            pltpu.VMEM((CG, HP), jnp.int32),
            pltpu.SemaphoreType.DMA,
            pltpu.SemaphoreType.DMA,
            pltpu.SemaphoreType.DMA,
            pltpu.SemaphoreType.DMA,
        ],
    )(_sc_gather_body)(ids_chunk, word_emb)


def _tc_ln_first(x_ref, pos_ref, type_ref, gamma_ref, beta_ref, o_ref):
    u = x_ref[...]
    fe = jax.lax.bitcast_convert_type(u << 16, jnp.float32)
    fo = jax.lax.bitcast_convert_type(u & jnp.int32(-65536), jnp.float32)
    x = (jnp.concatenate([fe, fo], axis=-1) + pos_ref[...]
         + type_ref[0:1, 0:1, :])
    m = jnp.mean(x, axis=-1, keepdims=True)
    d = x - m
    var = jnp.mean(d * d, axis=-1, keepdims=True)
    o = d * jax.lax.rsqrt(var + EPS)
    o_ref[...] = o * gamma_ref[0:1, 0:1, :] + beta_ref[0:1, 0:1, :]


def _tc_ln_acc(prev_ref, x_ref, pos_ref, type_ref, gamma_ref, beta_ref,
               o_ref):
    del prev_ref
    _tc_ln_first(x_ref, pos_ref, type_ref, gamma_ref, beta_ref, o_ref)


@jax.jit
def kernel(input_ids, word_emb, pos_emb, type_emb, gamma, beta):
    ids_flat = input_ids.reshape(-1).astype(jnp.int32)
    gm = gamma.reshape(1, HIDDEN)
    bt = beta.reshape(1, HIDDEN)

    w16 = word_emb.astype(jnp.bfloat16)
    word_packed = jax.lax.bitcast_convert_type(
        jnp.stack([w16[:, :HP], w16[:, HP:]], axis=-1), jnp.int32)
    gathered = [
        _sc_gather(lax.slice(ids_flat, (c * CHTOK,), ((c + 1) * CHTOK,)),
                   word_packed)
        for c in range(NCH)
    ]

    BS2 = 2
    pos3 = pos_emb.reshape(1, SEQ, HIDDEN)
    type3 = type_emb.reshape(1, 2, HIDDEN)
    gm3 = gm.reshape(1, 1, HIDDEN)
    bt3 = bt.reshape(1, 1, HIDDEN)
    common_in = [
        pl.BlockSpec((BS2, SEQ, HP), lambda i: (i, 0, 0)),
        pl.BlockSpec((1, SEQ, HIDDEN), lambda i: (0, 0, 0)),
        pl.BlockSpec((1, 2, HIDDEN), lambda i: (0, 0, 0)),
        pl.BlockSpec((1, 1, HIDDEN), lambda i: (0, 0, 0)),
        pl.BlockSpec((1, 1, HIDDEN), lambda i: (0, 0, 0)),
    ]
    out_shape = jax.ShapeDtypeStruct((BATCH, SEQ, HIDDEN), jnp.float32)

    out = pl.pallas_call(
        _tc_ln_first,
        grid=(CHB // BS2,),
        in_specs=common_in,
        out_specs=pl.BlockSpec((BS2, SEQ, HIDDEN), lambda i: (i, 0, 0)),
        out_shape=out_shape,
    )(gathered[0].reshape(CHB, SEQ, HP), pos3, type3, gm3, bt3)

    for c in range(1, NCH):
        out = pl.pallas_call(
            _tc_ln_acc,
            grid=(CHB // BS2,),
            in_specs=[pl.BlockSpec(memory_space=pl.ANY)] + common_in,
            out_specs=pl.BlockSpec(
                (BS2, SEQ, HIDDEN),
                lambda i, c=c: (c * (CHB // BS2) + i, 0, 0)),
            out_shape=out_shape,
            input_output_aliases={0: 0},
        )(out, gathered[c].reshape(CHB, SEQ, HP), pos3, type3, gm3, bt3)

    return out


# speedup vs baseline: 4.6268x; 1.0588x over previous
import functools

import jax
import jax.numpy as jnp
from jax import lax
from jax.experimental import pallas as pl
from jax.experimental.pallas import tpu as pltpu, tpu_sc as plsc

VOCAB = 30522
HIDDEN = 768
MAX_POS = 512
BATCH = 1024
SEQ = 512
EPS = 1e-12

NW = 32
NCH = 4
CHB = BATCH // NCH
CHTOK = CHB * SEQ
TOK_PER_W = CHTOK // NW
HP = HIDDEN // 2
CG = 64
NB = TOK_PER_W // CG


def _sc_gather_body(ids_hbm, word_hbm, out_hbm, idx_v, r0, r1,
                    sg0, sg1, ss0, ss1):
    wid = lax.axis_index("s") * 2 + lax.axis_index("c")
    base0 = wid * TOK_PER_W
    rows = (r0, r1)
    sgs = (sg0, sg1)
    sss = (ss0, ss1)

    pltpu.sync_copy(ids_hbm.at[pl.ds(base0, TOK_PER_W)], idx_v)
    pltpu.async_copy(word_hbm.at[idx_v.at[pl.ds(0, CG)]], r0, sg0)

    def pair(p, carry):
        for b in (0, 1):
            i = 2 * p + b
            pltpu.make_async_copy(
                word_hbm.at[idx_v.at[pl.ds(i * CG, CG)]], rows[b],
                sgs[b]).wait()

            pltpu.async_copy(rows[b], out_hbm.at[pl.ds(base0 + i * CG, CG)],
                             sss[b])

            @pl.when(jnp.logical_and(i >= 1, i + 1 < NB))
            def _issue_gather(b=b, i=i):
                pltpu.make_async_copy(rows[1 - b],
                                      out_hbm.at[pl.ds(0, CG)],
                                      sss[1 - b]).wait()
                pltpu.async_copy(
                    word_hbm.at[idx_v.at[pl.ds((i + 1) * CG, CG)]],
                    rows[1 - b], sgs[1 - b])

            @pl.when(i == 0)
            def _issue_gather0(b=b, i=i):
                pltpu.async_copy(
                    word_hbm.at[idx_v.at[pl.ds((i + 1) * CG, CG)]],
                    rows[1 - b], sgs[1 - b])
        return carry

    lax.fori_loop(0, NB // 2, pair, 0)
    pltpu.make_async_copy(r0, out_hbm.at[pl.ds(0, CG)], ss0).wait()
    pltpu.make_async_copy(r1, out_hbm.at[pl.ds(0, CG)], ss1).wait()


def _sc_gather(ids_chunk, word_emb):
    mesh = plsc.VectorSubcoreMesh(core_axis_name="c", subcore_axis_name="s")
    return functools.partial(
        pl.kernel,
        out_type=jax.ShapeDtypeStruct((CHTOK, HP), jnp.int32),
        mesh=mesh,
        compiler_params=pltpu.CompilerParams(needs_layout_passes=False),
        scratch_types=[
            pltpu.VMEM((TOK_PER_W,), jnp.int32),
            pltpu.VMEM((CG, HP), jnp.int32),
            pltpu.VMEM((CG, HP), jnp.int32),
            pltpu.SemaphoreType.DMA,
            pltpu.SemaphoreType.DMA,
            pltpu.SemaphoreType.DMA,
            pltpu.SemaphoreType.DMA,
        ],
    )(_sc_gather_body)(ids_chunk, word_emb)


def _tc_ln_first(x_ref, pos_ref, type_ref, gamma_ref, beta_ref, o_ref):
    u = x_ref[...]
    fe = jax.lax.bitcast_convert_type(u << 16, jnp.float32)
    fo = jax.lax.bitcast_convert_type(u & jnp.int32(-65536), jnp.float32)
    x = (jnp.concatenate([fe, fo], axis=-1) + pos_ref[...]
         + type_ref[0:1, 0:1, :])
    m = jnp.mean(x, axis=-1, keepdims=True)
    d = x - m
    var = jnp.mean(d * d, axis=-1, keepdims=True)
    o = d * jax.lax.rsqrt(var + EPS)
    o_ref[...] = o * gamma_ref[0:1, 0:1, :] + beta_ref[0:1, 0:1, :]


def _tc_ln_acc(prev_ref, x_ref, pos_ref, type_ref, gamma_ref, beta_ref,
               o_ref):
    del prev_ref
    _tc_ln_first(x_ref, pos_ref, type_ref, gamma_ref, beta_ref, o_ref)


@jax.jit
def kernel(input_ids, word_emb, pos_emb, type_emb, gamma, beta):
    ids_flat = input_ids.reshape(-1).astype(jnp.int32)
    gm = gamma.reshape(1, HIDDEN)
    bt = beta.reshape(1, HIDDEN)

    w16 = word_emb.astype(jnp.bfloat16)
    word_packed = jax.lax.bitcast_convert_type(
        jnp.stack([w16[:, :HP], w16[:, HP:]], axis=-1), jnp.int32)
    gathered = [
        _sc_gather(lax.slice(ids_flat, (c * CHTOK,), ((c + 1) * CHTOK,)),
                   word_packed)
        for c in range(NCH)
    ]

    BS2 = 4
    pos3 = pos_emb.reshape(1, SEQ, HIDDEN)
    type3 = type_emb.reshape(1, 2, HIDDEN)
    gm3 = gm.reshape(1, 1, HIDDEN)
    bt3 = bt.reshape(1, 1, HIDDEN)
    common_in = [
        pl.BlockSpec((BS2, SEQ, HP), lambda i: (i, 0, 0)),
        pl.BlockSpec((1, SEQ, HIDDEN), lambda i: (0, 0, 0)),
        pl.BlockSpec((1, 2, HIDDEN), lambda i: (0, 0, 0)),
        pl.BlockSpec((1, 1, HIDDEN), lambda i: (0, 0, 0)),
        pl.BlockSpec((1, 1, HIDDEN), lambda i: (0, 0, 0)),
    ]
    out_shape = jax.ShapeDtypeStruct((BATCH, SEQ, HIDDEN), jnp.float32)

    out = pl.pallas_call(
        _tc_ln_first,
        grid=(CHB // BS2,),
        in_specs=common_in,
        out_specs=pl.BlockSpec((BS2, SEQ, HIDDEN), lambda i: (i, 0, 0)),
        out_shape=out_shape,
    )(gathered[0].reshape(CHB, SEQ, HP), pos3, type3, gm3, bt3)

    for c in range(1, NCH):
        out = pl.pallas_call(
            _tc_ln_acc,
            grid=(CHB // BS2,),
            in_specs=[pl.BlockSpec(memory_space=pl.ANY)] + common_in,
            out_specs=pl.BlockSpec(
                (BS2, SEQ, HIDDEN),
                lambda i, c=c: (c * (CHB // BS2) + i, 0, 0)),
            out_shape=out_shape,
            input_output_aliases={0: 0},
        )(out, gathered[c].reshape(CHB, SEQ, HP), pos3, type3, gm3, bt3)

    return out


# speedup vs baseline: 4.7142x; 1.0189x over previous
import functools

import jax
import jax.numpy as jnp
from jax import lax
from jax.experimental import pallas as pl
from jax.experimental.pallas import tpu as pltpu, tpu_sc as plsc

VOCAB = 30522
HIDDEN = 768
MAX_POS = 512
BATCH = 1024
SEQ = 512
EPS = 1e-12

NW = 32
NCH = 4
CHB = BATCH // NCH
CHTOK = CHB * SEQ
TOK_PER_W = CHTOK // NW
HP = HIDDEN // 2
CG = 64
NB = TOK_PER_W // CG


def _sc_gather_body(ids_hbm, word_hbm, out_hbm, idx_v, r0, r1,
                    sg0, sg1, ss0, ss1):
    wid = lax.axis_index("s") * 2 + lax.axis_index("c")
    base0 = wid * TOK_PER_W
    rows = (r0, r1)
    sgs = (sg0, sg1)
    sss = (ss0, ss1)

    pltpu.sync_copy(ids_hbm.at[pl.ds(base0, TOK_PER_W)], idx_v)
    pltpu.async_copy(word_hbm.at[idx_v.at[pl.ds(0, CG)]], r0, sg0)

    def pair(p, carry):
        for b in (0, 1):
            i = 2 * p + b
            pltpu.make_async_copy(
                word_hbm.at[idx_v.at[pl.ds(i * CG, CG)]], rows[b],
                sgs[b]).wait()

            pltpu.async_copy(rows[b], out_hbm.at[pl.ds(base0 + i * CG, CG)],
                             sss[b])

            @pl.when(jnp.logical_and(i >= 1, i + 1 < NB))
            def _issue_gather(b=b, i=i):
                pltpu.make_async_copy(rows[1 - b],
                                      out_hbm.at[pl.ds(0, CG)],
                                      sss[1 - b]).wait()
                pltpu.async_copy(
                    word_hbm.at[idx_v.at[pl.ds((i + 1) * CG, CG)]],
                    rows[1 - b], sgs[1 - b])

            @pl.when(i == 0)
            def _issue_gather0(b=b, i=i):
                pltpu.async_copy(
                    word_hbm.at[idx_v.at[pl.ds((i + 1) * CG, CG)]],
                    rows[1 - b], sgs[1 - b])
        return carry

    lax.fori_loop(0, NB // 2, pair, 0)
    pltpu.make_async_copy(r0, out_hbm.at[pl.ds(0, CG)], ss0).wait()
    pltpu.make_async_copy(r1, out_hbm.at[pl.ds(0, CG)], ss1).wait()


def _sc_gather(ids_chunk, word_emb):
    mesh = plsc.VectorSubcoreMesh(core_axis_name="c", subcore_axis_name="s")
    return functools.partial(
        pl.kernel,
        out_type=jax.ShapeDtypeStruct((CHTOK, HP), jnp.int32),
        mesh=mesh,
        compiler_params=pltpu.CompilerParams(needs_layout_passes=False),
        scratch_types=[
            pltpu.VMEM((TOK_PER_W,), jnp.int32),
            pltpu.VMEM((CG, HP), jnp.int32),
            pltpu.VMEM((CG, HP), jnp.int32),
            pltpu.SemaphoreType.DMA,
            pltpu.SemaphoreType.DMA,
            pltpu.SemaphoreType.DMA,
            pltpu.SemaphoreType.DMA,
        ],
    )(_sc_gather_body)(ids_chunk, word_emb)


def _tc_ln_first(x_ref, pos_ref, type_ref, gamma_ref, beta_ref, o_ref):
    u = x_ref[...]
    fe = jax.lax.bitcast_convert_type(u << 16, jnp.float32)
    fo = jax.lax.bitcast_convert_type(u & jnp.int32(-65536), jnp.float32)
    x = (jnp.concatenate([fe, fo], axis=-1) + pos_ref[...]
         + type_ref[0:1, 0:1, :])
    m = jnp.mean(x, axis=-1, keepdims=True)
    d = x - m
    var = jnp.mean(d * d, axis=-1, keepdims=True)
    o = d * jax.lax.rsqrt(var + EPS)
    o_ref[...] = o * gamma_ref[0:1, 0:1, :] + beta_ref[0:1, 0:1, :]


def _tc_ln_acc(prev_ref, x_ref, pos_ref, type_ref, gamma_ref, beta_ref,
               o_ref):
    del prev_ref
    _tc_ln_first(x_ref, pos_ref, type_ref, gamma_ref, beta_ref, o_ref)


@jax.jit
def kernel(input_ids, word_emb, pos_emb, type_emb, gamma, beta):
    ids_flat = input_ids.reshape(-1).astype(jnp.int32)
    gm = gamma.reshape(1, HIDDEN)
    bt = beta.reshape(1, HIDDEN)

    w16 = word_emb.astype(jnp.bfloat16)
    word_packed = jax.lax.bitcast_convert_type(
        jnp.stack([w16[:, :HP], w16[:, HP:]], axis=-1), jnp.int32)
    gathered = [
        _sc_gather(lax.slice(ids_flat, (c * CHTOK,), ((c + 1) * CHTOK,)),
                   word_packed)
        for c in range(NCH)
    ]

    BS2 = 8
    pos3 = pos_emb.reshape(1, SEQ, HIDDEN)
    type3 = type_emb.reshape(1, 2, HIDDEN)
    gm3 = gm.reshape(1, 1, HIDDEN)
    bt3 = bt.reshape(1, 1, HIDDEN)
    common_in = [
        pl.BlockSpec((BS2, SEQ, HP), lambda i: (i, 0, 0)),
        pl.BlockSpec((1, SEQ, HIDDEN), lambda i: (0, 0, 0)),
        pl.BlockSpec((1, 2, HIDDEN), lambda i: (0, 0, 0)),
        pl.BlockSpec((1, 1, HIDDEN), lambda i: (0, 0, 0)),
        pl.BlockSpec((1, 1, HIDDEN), lambda i: (0, 0, 0)),
    ]
    out_shape = jax.ShapeDtypeStruct((BATCH, SEQ, HIDDEN), jnp.float32)

    out = pl.pallas_call(
        _tc_ln_first,
        grid=(CHB // BS2,),
        in_specs=common_in,
        out_specs=pl.BlockSpec((BS2, SEQ, HIDDEN), lambda i: (i, 0, 0)),
        out_shape=out_shape,
    )(gathered[0].reshape(CHB, SEQ, HP), pos3, type3, gm3, bt3)

    for c in range(1, NCH):
        out = pl.pallas_call(
            _tc_ln_acc,
            grid=(CHB // BS2,),
            in_specs=[pl.BlockSpec(memory_space=pl.ANY)] + common_in,
            out_specs=pl.BlockSpec(
                (BS2, SEQ, HIDDEN),
                lambda i, c=c: (c * (CHB // BS2) + i, 0, 0)),
            out_shape=out_shape,
            input_output_aliases={0: 0},
        )(out, gathered[c].reshape(CHB, SEQ, HP), pos3, type3, gm3, bt3)

    return out
